# baseline (speedup 1.0000x reference)
import functools

import jax
import jax.numpy as jnp
from jax import lax
from jax.experimental import pallas as pl
from jax.experimental.pallas import tpu as pltpu
from jax.experimental.pallas import tpu_sc as plsc

N = 25000
D = 128
E = 400000
NC = 2
NS = 16
DC = D // NC
B = 128
NBLK = 196
EPT = NBLK * B
EPAD = EPT * NS
NPAD = 25088
FB = 112
NFL = 14
RPT = NFL * FB


def _sc_body(eu0, ei0, rows, cols, vals,
             zu1, zi1, usum, isum,
             acc, fa, fb, gidx, sidx, vbuf, rowbuf, sem):
    c = lax.axis_index("c")
    s = lax.axis_index("s")
    iota = lax.iota(jnp.int32, 16)
    zeros16 = jnp.zeros((16,), jnp.float32)

    def zero_acc():
        @pl.loop(0, FB)
        def _(r):
            for k in range(DC // 16):
                fa[r, pl.ds(k * 16, 16)] = zeros16

        base = s * RPT
        for t in range(NFL):
            pltpu.sync_copy(fa, acc.at[pl.ds(base + t * FB, FB)])

    def edge_loop(tbl, g_hbm, s_hbm):
        ebase = s * EPT
        coff = c * NPAD

        @pl.loop(0, NBLK)
        def _(b):
            eoff = ebase + b * B
            pltpu.sync_copy(g_hbm.at[pl.ds(eoff, B)], gidx)
            pltpu.sync_copy(s_hbm.at[pl.ds(eoff, B)], sidx)
            pltpu.sync_copy(vals.at[pl.ds(eoff, B)], vbuf)
            for j in range(B // 16):
                gidx[pl.ds(j * 16, 16)] = gidx[pl.ds(j * 16, 16)] + coff
            pltpu.async_copy(tbl.at[gidx], rowbuf, sem).wait()

            @pl.loop(0, B // 16)
            def _(j):
                eidx = iota + j * 16
                ev = vbuf[pl.ds(j * 16, 16)]
                for k in range(DC):
                    col = jnp.full((16,), k, jnp.int32)
                    x = plsc.load_gather(rowbuf, [eidx, col])
                    plsc.store_scatter(rowbuf, [eidx, col], x * ev)

            pltpu.sync_copy(rowbuf, acc.at[sidx], add=True)

    def flush_raw(out):
        for t in range(NFL):
            r0 = s * RPT + t * FB
            pltpu.sync_copy(acc.at[pl.ds(r0, FB)], fa)
            pltpu.sync_copy(fa, out.at[pl.ds(c * NPAD + r0, FB)])

    def flush_combine(e0, z1, out):
        third = jnp.float32(1.0 / 3.0)
        for t in range(NFL):
            r0 = s * RPT + t * FB
            pltpu.sync_copy(acc.at[pl.ds(r0, FB)], fa)
            pltpu.sync_copy(e0.at[pl.ds(c * NPAD + r0, FB)], fb)

            @pl.loop(0, FB)
            def _(r):
                for k in range(DC // 16):
                    ds = pl.ds(k * 16, 16)
                    fa[r, ds] = fa[r, ds] + fb[r, ds]

            pltpu.sync_copy(z1.at[pl.ds(c * NPAD + r0, FB)], fb)

            @pl.loop(0, FB)
            def _(r):
                for k in range(DC // 16):
                    ds = pl.ds(k * 16, 16)
                    fa[r, ds] = (fa[r, ds] + fb[r, ds]) * third

            pltpu.sync_copy(fa, out.at[pl.ds(c * NPAD + r0, FB)])

    zero_acc()
    plsc.subcore_barrier()
    edge_loop(ei0, cols, rows)
    plsc.subcore_barrier()
    flush_raw(zu1)
    plsc.subcore_barrier()

    zero_acc()
    plsc.subcore_barrier()
    edge_loop(eu0, rows, cols)
    plsc.subcore_barrier()
    flush_raw(zi1)
    plsc.subcore_barrier()

    zero_acc()
    plsc.subcore_barrier()
    edge_loop(zi1, cols, rows)
    plsc.subcore_barrier()
    flush_combine(eu0, zu1, usum)
    plsc.subcore_barrier()

    zero_acc()
    plsc.subcore_barrier()
    edge_loop(zu1, rows, cols)
    plsc.subcore_barrier()
    flush_combine(ei0, zi1, isum)


_mesh = plsc.VectorSubcoreMesh(
    core_axis_name="c", subcore_axis_name="s", num_cores=NC, num_subcores=NS)

_tbl = jax.ShapeDtypeStruct((NC * NPAD, DC), jnp.float32)

_spmm = pl.kernel(
    _sc_body,
    out_type=(_tbl, _tbl, _tbl, _tbl),
    mesh=_mesh,
    compiler_params=pltpu.CompilerParams(
        needs_layout_passes=False, use_tc_tiling_on_sc=False),
    scratch_types=[
        pltpu.VMEM_SHARED((NPAD, DC), jnp.float32),
        pltpu.VMEM((FB, DC), jnp.float32),
        pltpu.VMEM((FB, DC), jnp.float32),
        pltpu.VMEM((B,), jnp.int32),
        pltpu.VMEM((B,), jnp.int32),
        pltpu.VMEM((B,), jnp.float32),
        pltpu.VMEM((B, DC), jnp.float32),
        pltpu.SemaphoreType.DMA,
    ],
)


def _to_chunked(x):
    xt = x.reshape(N, NC, DC).transpose(1, 0, 2)
    return jnp.pad(xt, ((0, 0), (0, NPAD - N), (0, 0))).reshape(NC * NPAD, DC)


def _from_chunked(x):
    return x.reshape(NC, NPAD, DC)[:, :N].transpose(1, 0, 2).reshape(N, D)


@jax.jit
def kernel(E_u_0, E_i_0, adj_indices, adj_values, u_mul_s, v_mul_s, ut, vt):
    rows = adj_indices[0].astype(jnp.int32)
    cols = adj_indices[1].astype(jnp.int32)
    vals = adj_values.astype(jnp.float32)
    pad = EPAD - E
    rows_p = jnp.concatenate([rows, jnp.zeros((pad,), jnp.int32)])
    cols_p = jnp.concatenate([cols, jnp.zeros((pad,), jnp.int32)])
    vals_p = jnp.concatenate([vals, jnp.zeros((pad,), jnp.float32)])
    eu0 = _to_chunked(E_u_0)
    ei0 = _to_chunked(E_i_0)
    _, _, us, it = _spmm(eu0, ei0, rows_p, cols_p, vals_p)
    return _from_chunked(us), _from_chunked(it)


# speedup vs baseline: 3.1202x; 3.1202x over previous
import functools

import jax
import jax.numpy as jnp
from jax import lax
from jax.experimental import pallas as pl
from jax.experimental.pallas import tpu as pltpu
from jax.experimental.pallas import tpu_sc as plsc

N = 25000
D = 128
E = 400000
NC = 2
NS = 16
DC = D // NC
B = 128
NBLK = 196
EPT = NBLK * B
EPAD = EPT * NS
NPAD = 25088
FB = 112
NFL = 14
RPT = NFL * FB


def _sc_body(eu0, ei0, rows, cols, vals,
             zu1, zi1, usum, isum,
             acc, fa, fb, gidx, sidx, vbuf, rowbuf, sem):
    c = lax.axis_index("c")
    s = lax.axis_index("s")
    iota = lax.iota(jnp.int32, 16)
    zeros16 = jnp.zeros((16,), jnp.float32)

    def zero_acc():
        @pl.loop(0, FB)
        def _(r):
            for k in range(DC // 16):
                fa[r, pl.ds(k * 16, 16)] = zeros16

        base = s * RPT
        for t in range(NFL):
            pltpu.sync_copy(fa, acc.at[pl.ds(base + t * FB, FB)])

    def edge_loop(tbl, g_hbm, s_hbm):
        ebase = s * EPT
        coff = c * NPAD

        @pl.loop(0, NBLK)
        def _(b):
            eoff = ebase + b * B
            pltpu.sync_copy(g_hbm.at[pl.ds(eoff, B)], gidx)
            pltpu.sync_copy(s_hbm.at[pl.ds(eoff, B)], sidx)
            pltpu.sync_copy(vals.at[pl.ds(eoff, B)], vbuf)
            for j in range(B // 16):
                gidx[pl.ds(j * 16, 16)] = gidx[pl.ds(j * 16, 16)] + coff
            pltpu.async_copy(tbl.at[gidx], rowbuf, sem).wait()

            @pl.loop(0, B, unroll=8)
            def _(e):
                ev = plsc.load_gather(vbuf, [jnp.zeros((16,), jnp.int32) + e])
                for k in range(DC // 16):
                    rowbuf[e, pl.ds(k * 16, 16)] = (
                        rowbuf[e, pl.ds(k * 16, 16)] * ev)

            pltpu.sync_copy(rowbuf, acc.at[sidx], add=True)

    def flush_raw(out):
        for t in range(NFL):
            r0 = s * RPT + t * FB
            pltpu.sync_copy(acc.at[pl.ds(r0, FB)], fa)
            pltpu.sync_copy(fa, out.at[pl.ds(c * NPAD + r0, FB)])

    def flush_combine(e0, z1, out):
        third = jnp.float32(1.0 / 3.0)
        for t in range(NFL):
            r0 = s * RPT + t * FB
            pltpu.sync_copy(acc.at[pl.ds(r0, FB)], fa)
            pltpu.sync_copy(e0.at[pl.ds(c * NPAD + r0, FB)], fb)

            @pl.loop(0, FB)
            def _(r):
                for k in range(DC // 16):
                    ds = pl.ds(k * 16, 16)
                    fa[r, ds] = fa[r, ds] + fb[r, ds]

            pltpu.sync_copy(z1.at[pl.ds(c * NPAD + r0, FB)], fb)

            @pl.loop(0, FB)
            def _(r):
                for k in range(DC // 16):
                    ds = pl.ds(k * 16, 16)
                    fa[r, ds] = (fa[r, ds] + fb[r, ds]) * third

            pltpu.sync_copy(fa, out.at[pl.ds(c * NPAD + r0, FB)])

    zero_acc()
    plsc.subcore_barrier()
    edge_loop(ei0, cols, rows)
    plsc.subcore_barrier()
    flush_raw(zu1)
    plsc.subcore_barrier()

    zero_acc()
    plsc.subcore_barrier()
    edge_loop(eu0, rows, cols)
    plsc.subcore_barrier()
    flush_raw(zi1)
    plsc.subcore_barrier()

    zero_acc()
    plsc.subcore_barrier()
    edge_loop(zi1, cols, rows)
    plsc.subcore_barrier()
    flush_combine(eu0, zu1, usum)
    plsc.subcore_barrier()

    zero_acc()
    plsc.subcore_barrier()
    edge_loop(zu1, rows, cols)
    plsc.subcore_barrier()
    flush_combine(ei0, zi1, isum)


_mesh = plsc.VectorSubcoreMesh(
    core_axis_name="c", subcore_axis_name="s", num_cores=NC, num_subcores=NS)

_tbl = jax.ShapeDtypeStruct((NC * NPAD, DC), jnp.float32)

_spmm = pl.kernel(
    _sc_body,
    out_type=(_tbl, _tbl, _tbl, _tbl),
    mesh=_mesh,
    compiler_params=pltpu.CompilerParams(
        needs_layout_passes=False, use_tc_tiling_on_sc=False),
    scratch_types=[
        pltpu.VMEM_SHARED((NPAD, DC), jnp.float32),
        pltpu.VMEM((FB, DC), jnp.float32),
        pltpu.VMEM((FB, DC), jnp.float32),
        pltpu.VMEM((B,), jnp.int32),
        pltpu.VMEM((B,), jnp.int32),
        pltpu.VMEM((B,), jnp.float32),
        pltpu.VMEM((B, DC), jnp.float32),
        pltpu.SemaphoreType.DMA,
    ],
)


def _to_chunked(x):
    xt = x.reshape(N, NC, DC).transpose(1, 0, 2)
    return jnp.pad(xt, ((0, 0), (0, NPAD - N), (0, 0))).reshape(NC * NPAD, DC)


def _from_chunked(x):
    return x.reshape(NC, NPAD, DC)[:, :N].transpose(1, 0, 2).reshape(N, D)


@jax.jit
def kernel(E_u_0, E_i_0, adj_indices, adj_values, u_mul_s, v_mul_s, ut, vt):
    rows = adj_indices[0].astype(jnp.int32)
    cols = adj_indices[1].astype(jnp.int32)
    vals = adj_values.astype(jnp.float32)
    pad = EPAD - E
    rows_p = jnp.concatenate([rows, jnp.zeros((pad,), jnp.int32)])
    cols_p = jnp.concatenate([cols, jnp.zeros((pad,), jnp.int32)])
    vals_p = jnp.concatenate([vals, jnp.zeros((pad,), jnp.float32)])
    eu0 = _to_chunked(E_u_0)
    ei0 = _to_chunked(E_i_0)
    _, _, us, it = _spmm(eu0, ei0, rows_p, cols_p, vals_p)
    return _from_chunked(us), _from_chunked(it)


# speedup vs baseline: 3.9497x; 1.2658x over previous
import jax
import jax.numpy as jnp
from jax import lax
from jax.experimental import pallas as pl
from jax.experimental.pallas import tpu as pltpu
from jax.experimental.pallas import tpu_sc as plsc

N = 25000
D = 128
E = 400000
NC = 2
NS = 16
DC = D // NC
B = 64
NBLK = 392
EPT = NBLK * B
EPAD = EPT * NS
SBB = 56
SBN = NBLK // SBB
S = SBB * B
NPAD = 25600
FB = 64
NFL = 25
RPT = NFL * FB


def _sc_body(eu0, ei0, rows2, cols2, vals2,
             zu1, zi1, usum, isum,
             acc, gbuf0, gbuf1, sbuf0, sbuf1, gidx, sidx2, vbuf,
             g0, g1, s0, s1):
    c = lax.axis_index("c")
    s = lax.axis_index("s")
    zeros16 = jnp.zeros((16,), jnp.float32)
    zeros16i = jnp.zeros((16,), jnp.int32)
    coff = c * NPAD

    def zero_acc():
        @pl.loop(0, FB, unroll=4)
        def _(r):
            for k in range(DC // 16):
                gbuf0[r, pl.ds(k * 16, 16)] = zeros16

        base = s * RPT

        @pl.loop(0, NFL)
        def _(t):
            pltpu.async_copy(gbuf0, acc.at[pl.ds(base + t * FB, FB)], g0)

        @pl.loop(0, NFL)
        def _(t):
            pltpu.make_async_copy(
                gbuf0, acc.at[pl.ds(base + t * FB, FB)], g0).wait()

    def edge_loop(tbl, g2d_hbm, s2d_hbm):
        @pl.loop(0, SBN)
        def _(sb):
            rbase = s * NBLK + sb * SBB
            pltpu.sync_copy(g2d_hbm.at[pl.ds(rbase, SBB)], gidx)
            pltpu.sync_copy(s2d_hbm.at[pl.ds(rbase, SBB)], sidx2)
            pltpu.sync_copy(vals2.at[pl.ds(rbase, SBB)], vbuf)

            @pl.loop(0, SBB, unroll=4)
            def _(j):
                for k in range(B // 16):
                    ds = pl.ds(k * 16, 16)
                    gidx[j, ds] = gidx[j, ds] + coff

            pltpu.async_copy(tbl.at[gidx.at[0]], gbuf0, g0)
            pltpu.async_copy(tbl.at[gidx.at[1]], gbuf1, g1)

            @pl.loop(0, SBB // 2)
            def _(jj):
                for gbuf, sbuf, gsem, ssem, par in (
                        (gbuf0, sbuf0, g0, s0, 0), (gbuf1, sbuf1, g1, s1, 1)):
                    m = jj * 2 + par
                    pltpu.make_async_copy(
                        tbl.at[gidx.at[m]], gbuf, gsem).wait()

                    @pl.when(jj > 0)
                    def _():
                        pltpu.make_async_copy(
                            sbuf, acc.at[sidx2.at[m - 2]], ssem).wait()

                    @pl.loop(0, B, unroll=4)
                    def _(e):
                        ev = plsc.load_gather(
                            vbuf, [zeros16i + m, zeros16i + e])
                        for k in range(DC // 16):
                            ds = pl.ds(k * 16, 16)
                            sbuf[e, ds] = gbuf[e, ds] * ev

                    pltpu.async_copy(sbuf, acc.at[sidx2.at[m]], ssem, add=True)

                    @pl.when(jj < SBB // 2 - 1)
                    def _():
                        pltpu.async_copy(tbl.at[gidx.at[m + 2]], gbuf, gsem)

            pltpu.make_async_copy(sbuf0, acc.at[sidx2.at[SBB - 2]], s0).wait()
            pltpu.make_async_copy(sbuf1, acc.at[sidx2.at[SBB - 1]], s1).wait()

    def flush_raw(out):
        @pl.loop(0, NFL)
        def _(t):
            r0 = s * RPT + t * FB
            pltpu.sync_copy(acc.at[pl.ds(r0, FB)], gbuf0)
            pltpu.sync_copy(gbuf0, out.at[pl.ds(coff + r0, FB)])

    def flush_combine(e0, z1, out):
        third = jnp.float32(1.0 / 3.0)

        @pl.loop(0, NFL)
        def _(t):
            r0 = s * RPT + t * FB
            pltpu.sync_copy(acc.at[pl.ds(r0, FB)], gbuf0)
            pltpu.sync_copy(e0.at[pl.ds(coff + r0, FB)], gbuf1)

            @pl.loop(0, FB, unroll=4)
            def _(r):
                for k in range(DC // 16):
                    ds = pl.ds(k * 16, 16)
                    gbuf0[r, ds] = gbuf0[r, ds] + gbuf1[r, ds]

            pltpu.sync_copy(z1.at[pl.ds(coff + r0, FB)], gbuf1)

            @pl.loop(0, FB, unroll=4)
            def _(r):
                for k in range(DC // 16):
                    ds = pl.ds(k * 16, 16)
                    gbuf0[r, ds] = (gbuf0[r, ds] + gbuf1[r, ds]) * third

            pltpu.sync_copy(gbuf0, out.at[pl.ds(coff + r0, FB)])

    zero_acc()
    plsc.subcore_barrier()
    edge_loop(ei0, cols2, rows2)
    plsc.subcore_barrier()
    flush_raw(zu1)
    plsc.subcore_barrier()

    zero_acc()
    plsc.subcore_barrier()
    edge_loop(eu0, rows2, cols2)
    plsc.subcore_barrier()
    flush_raw(zi1)
    plsc.subcore_barrier()

    zero_acc()
    plsc.subcore_barrier()
    edge_loop(zi1, cols2, rows2)
    plsc.subcore_barrier()
    flush_combine(eu0, zu1, usum)
    plsc.subcore_barrier()

    zero_acc()
    plsc.subcore_barrier()
    edge_loop(zu1, rows2, cols2)
    plsc.subcore_barrier()
    flush_combine(ei0, zi1, isum)


_mesh = plsc.VectorSubcoreMesh(
    core_axis_name="c", subcore_axis_name="s", num_cores=NC, num_subcores=NS)

_tbl = jax.ShapeDtypeStruct((NC * NPAD, DC), jnp.float32)

_spmm = pl.kernel(
    _sc_body,
    out_type=(_tbl, _tbl, _tbl, _tbl),
    mesh=_mesh,
    compiler_params=pltpu.CompilerParams(
        needs_layout_passes=False, use_tc_tiling_on_sc=False),
    scratch_types=[
        pltpu.VMEM_SHARED((NPAD, DC), jnp.float32),
        pltpu.VMEM((B, DC), jnp.float32),
        pltpu.VMEM((B, DC), jnp.float32),
        pltpu.VMEM((B, DC), jnp.float32),
        pltpu.VMEM((B, DC), jnp.float32),
        pltpu.VMEM((SBB, B), jnp.int32),
        pltpu.VMEM((SBB, B), jnp.int32),
        pltpu.VMEM((SBB, B), jnp.float32),
        pltpu.SemaphoreType.DMA,
        pltpu.SemaphoreType.DMA,
        pltpu.SemaphoreType.DMA,
        pltpu.SemaphoreType.DMA,
    ],
)


def _to_chunked(x):
    xt = x.reshape(N, NC, DC).transpose(1, 0, 2)
    return jnp.pad(xt, ((0, 0), (0, NPAD - N), (0, 0))).reshape(NC * NPAD, DC)


def _from_chunked(x):
    return x.reshape(NC, NPAD, DC)[:, :N].transpose(1, 0, 2).reshape(N, D)


@jax.jit
def kernel(E_u_0, E_i_0, adj_indices, adj_values, u_mul_s, v_mul_s, ut, vt):
    rows = adj_indices[0].astype(jnp.int32)
    cols = adj_indices[1].astype(jnp.int32)
    vals = adj_values.astype(jnp.float32)
    pad = EPAD - E
    rows_p = jnp.concatenate(
        [rows, jnp.zeros((pad,), jnp.int32)]).reshape(EPAD // B, B)
    cols_p = jnp.concatenate(
        [cols, jnp.zeros((pad,), jnp.int32)]).reshape(EPAD // B, B)
    vals_p = jnp.concatenate(
        [vals, jnp.zeros((pad,), jnp.float32)]).reshape(EPAD // B, B)
    eu0 = _to_chunked(E_u_0)
    ei0 = _to_chunked(E_i_0)
    _, _, us, it = _spmm(eu0, ei0, rows_p, cols_p, vals_p)
    return _from_chunked(us), _from_chunked(it)


# speedup vs baseline: 6.8703x; 1.7394x over previous
import jax
import jax.numpy as jnp
from jax import lax
from jax.experimental import pallas as pl
from jax.experimental.pallas import tpu as pltpu
from jax.experimental.pallas import tpu_sc as plsc

N = 25000
D = 128
E = 400000
NC = 2
NS = 16
DC = D // NC
B = 64
NBLK = 392
EPT = NBLK * B
EPAD = EPT * NS
SBB = 56
SBN = NBLK // SBB
S = SBB * B
NPAD = 25600
FB = 64
NFL = 25
RPT = NFL * FB


_GDN = lax.GatherDimensionNumbers(
    offset_dims=(), collapsed_slice_dims=(0,), start_index_map=(0,))


def _lane_broadcast(v, lane):
    idx = jnp.full((16, 1), lane, jnp.int32)
    return lax.gather(v, idx, dimension_numbers=_GDN, slice_sizes=(1,),
                      mode=lax.GatherScatterMode.PROMISE_IN_BOUNDS)


def _sc_body(eu0, ei0, rows2, cols2, vals2,
             zu1, zi1, usum, isum,
             acc, gbuf0, gbuf1, sbuf0, sbuf1, gidx, sidx2, vbuf,
             g0, g1, s0, s1):
    c = lax.axis_index("c")
    s = lax.axis_index("s")
    zeros16 = jnp.zeros((16,), jnp.float32)
    zeros16i = jnp.zeros((16,), jnp.int32)
    coff = c * NPAD

    def zero_acc():
        @pl.loop(0, FB, unroll=4)
        def _(r):
            for k in range(DC // 16):
                gbuf0[r, pl.ds(k * 16, 16)] = zeros16

        base = s * RPT

        @pl.loop(0, NFL)
        def _(t):
            pltpu.async_copy(gbuf0, acc.at[pl.ds(base + t * FB, FB)], g0)

        @pl.loop(0, NFL)
        def _(t):
            pltpu.make_async_copy(
                gbuf0, acc.at[pl.ds(base + t * FB, FB)], g0).wait()

    def edge_loop(tbl, g2d_hbm, s2d_hbm):
        @pl.loop(0, SBN)
        def _(sb):
            rbase = s * NBLK + sb * SBB
            pltpu.sync_copy(g2d_hbm.at[pl.ds(rbase, SBB)], gidx)
            pltpu.sync_copy(s2d_hbm.at[pl.ds(rbase, SBB)], sidx2)
            pltpu.sync_copy(vals2.at[pl.ds(rbase, SBB)], vbuf)

            @pl.loop(0, SBB, unroll=4)
            def _(j):
                for k in range(B // 16):
                    ds = pl.ds(k * 16, 16)
                    gidx[j, ds] = gidx[j, ds] + coff

            pltpu.async_copy(tbl.at[gidx.at[0]], gbuf0, g0)
            pltpu.async_copy(tbl.at[gidx.at[1]], gbuf1, g1)

            @pl.loop(0, SBB // 2)
            def _(jj):
                for gbuf, sbuf, gsem, ssem, par in (
                        (gbuf0, sbuf0, g0, s0, 0), (gbuf1, sbuf1, g1, s1, 1)):
                    m = jj * 2 + par
                    pltpu.make_async_copy(
                        tbl.at[gidx.at[m]], gbuf, gsem).wait()

                    @pl.when(jj > 0)
                    def _():
                        pltpu.make_async_copy(
                            sbuf, acc.at[sidx2.at[m - 2]], ssem).wait()

                    @pl.loop(0, B // 16)
                    def _(g):
                        vv = vbuf[m, pl.ds(g * 16, 16)]
                        for e16 in range(16):
                            ev = _lane_broadcast(vv, e16)
                            e = g * 16 + e16
                            for k in range(DC // 16):
                                ds = pl.ds(k * 16, 16)
                                sbuf[e, ds] = gbuf[e, ds] * ev

                    pltpu.async_copy(sbuf, acc.at[sidx2.at[m]], ssem, add=True)

                    @pl.when(jj < SBB // 2 - 1)
                    def _():
                        pltpu.async_copy(tbl.at[gidx.at[m + 2]], gbuf, gsem)

            pltpu.make_async_copy(sbuf0, acc.at[sidx2.at[SBB - 2]], s0).wait()
            pltpu.make_async_copy(sbuf1, acc.at[sidx2.at[SBB - 1]], s1).wait()

    def flush_raw(out):
        @pl.loop(0, NFL)
        def _(t):
            r0 = s * RPT + t * FB
            pltpu.sync_copy(acc.at[pl.ds(r0, FB)], gbuf0)
            pltpu.sync_copy(gbuf0, out.at[pl.ds(coff + r0, FB)])

    def flush_combine(e0, z1, out):
        third = jnp.float32(1.0 / 3.0)

        @pl.loop(0, NFL)
        def _(t):
            r0 = s * RPT + t * FB
            pltpu.sync_copy(acc.at[pl.ds(r0, FB)], gbuf0)
            pltpu.sync_copy(e0.at[pl.ds(coff + r0, FB)], gbuf1)

            @pl.loop(0, FB, unroll=4)
            def _(r):
                for k in range(DC // 16):
                    ds = pl.ds(k * 16, 16)
                    gbuf0[r, ds] = gbuf0[r, ds] + gbuf1[r, ds]

            pltpu.sync_copy(z1.at[pl.ds(coff + r0, FB)], gbuf1)

            @pl.loop(0, FB, unroll=4)
            def _(r):
                for k in range(DC // 16):
                    ds = pl.ds(k * 16, 16)
                    gbuf0[r, ds] = (gbuf0[r, ds] + gbuf1[r, ds]) * third

            pltpu.sync_copy(gbuf0, out.at[pl.ds(coff + r0, FB)])

    zero_acc()
    plsc.subcore_barrier()
    edge_loop(ei0, cols2, rows2)
    plsc.subcore_barrier()
    flush_raw(zu1)
    plsc.subcore_barrier()

    zero_acc()
    plsc.subcore_barrier()
    edge_loop(eu0, rows2, cols2)
    plsc.subcore_barrier()
    flush_raw(zi1)
    plsc.subcore_barrier()

    zero_acc()
    plsc.subcore_barrier()
    edge_loop(zi1, cols2, rows2)
    plsc.subcore_barrier()
    flush_combine(eu0, zu1, usum)
    plsc.subcore_barrier()

    zero_acc()
    plsc.subcore_barrier()
    edge_loop(zu1, rows2, cols2)
    plsc.subcore_barrier()
    flush_combine(ei0, zi1, isum)


_mesh = plsc.VectorSubcoreMesh(
    core_axis_name="c", subcore_axis_name="s", num_cores=NC, num_subcores=NS)

_tbl = jax.ShapeDtypeStruct((NC * NPAD, DC), jnp.float32)

_spmm = pl.kernel(
    _sc_body,
    out_type=(_tbl, _tbl, _tbl, _tbl),
    mesh=_mesh,
    compiler_params=pltpu.CompilerParams(
        needs_layout_passes=False, use_tc_tiling_on_sc=False),
    scratch_types=[
        pltpu.VMEM_SHARED((NPAD, DC), jnp.float32),
        pltpu.VMEM((B, DC), jnp.float32),
        pltpu.VMEM((B, DC), jnp.float32),
        pltpu.VMEM((B, DC), jnp.float32),
        pltpu.VMEM((B, DC), jnp.float32),
        pltpu.VMEM((SBB, B), jnp.int32),
        pltpu.VMEM((SBB, B), jnp.int32),
        pltpu.VMEM((SBB, B), jnp.float32),
        pltpu.SemaphoreType.DMA,
        pltpu.SemaphoreType.DMA,
        pltpu.SemaphoreType.DMA,
        pltpu.SemaphoreType.DMA,
    ],
)


def _to_chunked(x):
    xt = x.reshape(N, NC, DC).transpose(1, 0, 2)
    return jnp.pad(xt, ((0, 0), (0, NPAD - N), (0, 0))).reshape(NC * NPAD, DC)


def _from_chunked(x):
    return x.reshape(NC, NPAD, DC)[:, :N].transpose(1, 0, 2).reshape(N, D)


@jax.jit
def kernel(E_u_0, E_i_0, adj_indices, adj_values, u_mul_s, v_mul_s, ut, vt):
    rows = adj_indices[0].astype(jnp.int32)
    cols = adj_indices[1].astype(jnp.int32)
    vals = adj_values.astype(jnp.float32)
    pad = EPAD - E
    rows_p = jnp.concatenate(
        [rows, jnp.zeros((pad,), jnp.int32)]).reshape(EPAD // B, B)
    cols_p = jnp.concatenate(
        [cols, jnp.zeros((pad,), jnp.int32)]).reshape(EPAD // B, B)
    vals_p = jnp.concatenate(
        [vals, jnp.zeros((pad,), jnp.float32)]).reshape(EPAD // B, B)
    eu0 = _to_chunked(E_u_0)
    ei0 = _to_chunked(E_i_0)
    _, _, us, it = _spmm(eu0, ei0, rows_p, cols_p, vals_p)
    return _from_chunked(us), _from_chunked(it)
